# baseline (speedup 1.0000x reference)
import functools

import jax
import jax.numpy as jnp
from jax import lax
from jax.experimental import pallas as pl
from jax.experimental.pallas import tpu as pltpu
from jax.experimental.pallas import tpu_sc as plsc

_N = 131072
_QLEN = 8192
_NBLK = 64
_BS = 128
_DEST = _QLEN * _NBLK

_NCHUNK = 4
_NRANGE = 8
_CHUNK = _N // _NCHUNK
_RANGE = _DEST // _NRANGE
_PIECE = 4096
_NPIECE = _CHUNK // _PIECE
_UNROLL = 8

_BT = 128


def _sc_winner(packed):
    mesh = plsc.VectorSubcoreMesh(core_axis_name="c", subcore_axis_name="s")

    @functools.partial(
        pl.kernel,
        mesh=mesh,
        compiler_params=pltpu.CompilerParams(needs_layout_passes=False),
        out_type=jax.ShapeDtypeStruct((_NCHUNK, _DEST), jnp.int32),
        scratch_types=[
            pltpu.VMEM((_RANGE,), jnp.int32),
            pltpu.VMEM((2, _PIECE), jnp.int32),
            pltpu.SemaphoreType.DMA,
            pltpu.SemaphoreType.DMA,
        ],
    )
    def run(pk_hbm, out_hbm, w_v, pk_v, sem0, sem1):
        wid = lax.axis_index("s") * 2 + lax.axis_index("c")
        u = wid // _NRANGE
        r = wid % _NRANGE
        lo = r * _RANGE
        sems = (sem0, sem1)

        def start_piece(p):
            s = p % 2
            base = u * _CHUNK + p * _PIECE
            return pltpu.async_copy(
                pk_hbm.at[pl.ds(base, _PIECE)], pk_v.at[s], sems[s])

        handles = {0: start_piece(0), 1: start_piece(1)}

        def init_body(i, carry):
            b = i * (4 * 16)
            w_v[pl.ds(b, 16)] = jnp.full((16,), -1, jnp.int32)
            w_v[pl.ds(b + 16, 16)] = jnp.full((16,), -1, jnp.int32)
            w_v[pl.ds(b + 32, 16)] = jnp.full((16,), -1, jnp.int32)
            w_v[pl.ds(b + 48, 16)] = jnp.full((16,), -1, jnp.int32)
            return carry

        lax.fori_loop(0, _RANGE // 64, init_body, 0)

        def process(s, j):
            pk = pk_v[s, pl.ds(j * 16, 16)]
            d = lax.shift_right_arithmetic(pk, 8) - lo
            inr = (d >= 0) & (d < _RANGE)
            plsc.store_scatter(w_v, [d], pk, mask=inr)

        for p in range(_NPIECE):
            s = p % 2
            handles.pop(p).wait()

            def vec_body(i, carry, s=s):
                for k in range(_UNROLL):
                    process(s, i * _UNROLL + k)
                return carry

            lax.fori_loop(0, _PIECE // (16 * _UNROLL), vec_body, 0)
            if p + 2 < _NPIECE:
                handles[p + 2] = start_piece(p + 2)

        pltpu.sync_copy(w_v, out_hbm.at[u, pl.ds(lo, _RANGE)])

    return run(packed)


def _tc_expand(w_all):
    def body(w_ref, o_ref):
        w = w_ref[...]
        winner = jnp.where(
            w[3] >= 0, w[3],
            jnp.where(w[2] >= 0, w[2],
                      jnp.where(w[1] >= 0, w[1],
                                jnp.maximum(w[0], 0))))
        usage = winner & 255
        c = lax.broadcasted_iota(jnp.int32, (_BT, _NBLK, _BS), 2)
        o_ref[...] = jnp.where(c + 1 > usage[:, :, None],
                               jnp.float32(-jnp.inf), jnp.float32(0.0))

    return pl.pallas_call(
        body,
        grid=(_QLEN // _BT,),
        in_specs=[pl.BlockSpec((_NCHUNK, _BT, _NBLK), lambda i: (0, i, 0))],
        out_specs=pl.BlockSpec((_BT, _NBLK, _BS), lambda i: (i, 0, 0)),
        out_shape=jax.ShapeDtypeStruct((_QLEN, _NBLK, _BS), jnp.float32),
    )(w_all)


def kernel(block_usages, hpu_shared_token_idx, hpu_shared_block_idx,
           block_size, target_qlen, target_shared_blocks):
    bdelta = (jnp.asarray(block_size) - _BS).astype(jnp.float32)
    qdelta = (jnp.asarray(target_qlen) - _QLEN).astype(hpu_shared_token_idx.dtype)
    sdelta = (jnp.asarray(target_shared_blocks) - _NBLK).astype(hpu_shared_block_idx.dtype)
    usage = jnp.clip(block_usages.astype(jnp.float32) - bdelta, 0.0, float(_BS))
    tok = (hpu_shared_token_idx + qdelta).astype(jnp.int32)
    blk = (hpu_shared_block_idx + sdelta).astype(jnp.int32)
    packed = (tok * _NBLK + blk) * 256 + usage.astype(jnp.int32)

    w_all = _sc_winner(packed)
    return _tc_expand(w_all.reshape(_NCHUNK, _QLEN, _NBLK))


# speedup vs baseline: 1.1251x; 1.1251x over previous
import functools

import jax
import jax.numpy as jnp
from jax import lax
from jax.experimental import pallas as pl
from jax.experimental.pallas import tpu as pltpu
from jax.experimental.pallas import tpu_sc as plsc

_N = 131072
_QLEN = 8192
_NBLK = 64
_BS = 128
_DEST = _QLEN * _NBLK

_NCHUNK = 4
_NRANGE = 8
_CHUNK = _N // _NCHUNK
_RANGE = _DEST // _NRANGE
_PIECE = 8192
_NPIECE = _CHUNK // _PIECE
_UNROLL = 8

_BT = 512


def _sc_winner(packed):
    mesh = plsc.VectorSubcoreMesh(core_axis_name="c", subcore_axis_name="s")

    @functools.partial(
        pl.kernel,
        mesh=mesh,
        compiler_params=pltpu.CompilerParams(needs_layout_passes=False),
        out_type=jax.ShapeDtypeStruct((_NCHUNK, _DEST), jnp.int32),
        scratch_types=[
            pltpu.VMEM((_RANGE,), jnp.int32),
            pltpu.VMEM((2, _PIECE), jnp.int32),
            pltpu.SemaphoreType.DMA,
            pltpu.SemaphoreType.DMA,
        ],
    )
    def run(pk_hbm, out_hbm, w_v, pk_v, sem0, sem1):
        wid = lax.axis_index("s") * 2 + lax.axis_index("c")
        u = wid // _NRANGE
        r = wid % _NRANGE
        lo = r * _RANGE
        sems = (sem0, sem1)

        def start_piece(p):
            s = p % 2
            base = u * _CHUNK + p * _PIECE
            return pltpu.async_copy(
                pk_hbm.at[pl.ds(base, _PIECE)], pk_v.at[s], sems[s])

        handles = {0: start_piece(0), 1: start_piece(1)}

        def init_body(i, carry):
            b = i * (4 * 16)
            w_v[pl.ds(b, 16)] = jnp.full((16,), -1, jnp.int32)
            w_v[pl.ds(b + 16, 16)] = jnp.full((16,), -1, jnp.int32)
            w_v[pl.ds(b + 32, 16)] = jnp.full((16,), -1, jnp.int32)
            w_v[pl.ds(b + 48, 16)] = jnp.full((16,), -1, jnp.int32)
            return carry

        lax.fori_loop(0, _RANGE // 64, init_body, 0)

        def process(s, j):
            pk = pk_v[s, pl.ds(j * 16, 16)]
            d = lax.shift_right_arithmetic(pk, 8) - lo
            inr = (d >= 0) & (d < _RANGE)
            plsc.store_scatter(w_v, [d], pk, mask=inr)

        for p in range(_NPIECE):
            s = p % 2
            handles.pop(p).wait()

            def vec_body(i, carry, s=s):
                for k in range(_UNROLL):
                    process(s, i * _UNROLL + k)
                return carry

            lax.fori_loop(0, _PIECE // (16 * _UNROLL), vec_body, 0)
            if p + 2 < _NPIECE:
                handles[p + 2] = start_piece(p + 2)

        pltpu.sync_copy(w_v, out_hbm.at[u, pl.ds(lo, _RANGE)])

    return run(packed)


def _tc_expand(w_all):
    def body(w_ref, o_ref):
        w = w_ref[...]
        winner = jnp.where(
            w[3] >= 0, w[3],
            jnp.where(w[2] >= 0, w[2],
                      jnp.where(w[1] >= 0, w[1],
                                jnp.maximum(w[0], 0))))
        usage = winner & 255
        c = lax.broadcasted_iota(jnp.int32, (_BT, _NBLK, _BS), 2)
        o_ref[...] = jnp.where(c + 1 > usage[:, :, None],
                               jnp.float32(-jnp.inf), jnp.float32(0.0))

    return pl.pallas_call(
        body,
        grid=(_QLEN // _BT,),
        in_specs=[pl.BlockSpec((_NCHUNK, _BT, _NBLK), lambda i: (0, i, 0))],
        out_specs=pl.BlockSpec((_BT, _NBLK, _BS), lambda i: (i, 0, 0)),
        out_shape=jax.ShapeDtypeStruct((_QLEN, _NBLK, _BS), jnp.float32),
    )(w_all)


def kernel(block_usages, hpu_shared_token_idx, hpu_shared_block_idx,
           block_size, target_qlen, target_shared_blocks):
    bdelta = (jnp.asarray(block_size) - _BS).astype(jnp.float32)
    qdelta = (jnp.asarray(target_qlen) - _QLEN).astype(hpu_shared_token_idx.dtype)
    sdelta = (jnp.asarray(target_shared_blocks) - _NBLK).astype(hpu_shared_block_idx.dtype)
    usage = jnp.clip(block_usages.astype(jnp.float32) - bdelta, 0.0, float(_BS))
    tok = (hpu_shared_token_idx + qdelta).astype(jnp.int32)
    blk = (hpu_shared_block_idx + sdelta).astype(jnp.int32)
    packed = (tok * _NBLK + blk) * 256 + usage.astype(jnp.int32)

    w_all = _sc_winner(packed)
    return _tc_expand(w_all.reshape(_NCHUNK, _QLEN, _NBLK))
